# Initial kernel scaffold; baseline (speedup 1.0000x reference)
import jax
import jax.numpy as jnp
from jax.experimental import pallas as pl


def kernel(buckets, node_ids, token_ids):
    raise NotImplementedError("write your pallas kernel here")


# speedup vs baseline: 6.5619x; 6.5619x over previous
import functools

import jax
import jax.numpy as jnp
from jax import lax
from jax.experimental import pallas as pl
from jax.experimental.pallas import tpu as pltpu
from jax.experimental.pallas import tpu_sc as plsc

D = 128
B = 16384
T = 20

NC = 2
NS = 16
NW = NC * NS
NPW = B // NW
ROWS_W = NPW * T
IDX_ROWS = ROWS_W // 128
CHUNK_NODES = 32
CHUNK_ROWS = CHUNK_NODES * T
CHUNK_IDX = CHUNK_ROWS // 128
N_CHUNKS = NPW // CHUNK_NODES


def _node_embed_sc(buckets, tok):
    mesh = plsc.VectorSubcoreMesh(core_axis_name="c", subcore_axis_name="s")

    @functools.partial(
        pl.kernel,
        mesh=mesh,
        out_type=jax.ShapeDtypeStruct((B, D), jnp.float32),
        scratch_types=[
            pltpu.VMEM((IDX_ROWS, 128), jnp.int32),
            pltpu.VMEM((CHUNK_ROWS, D), jnp.float32),
            pltpu.VMEM((CHUNK_NODES, D), jnp.float32),
            pltpu.SemaphoreType.DMA,
        ],
    )
    def k(table_hbm, tok_hbm, out_hbm, idx_v, rows_v, out_v, sem):
        i32 = jnp.int32
        wid = lax.axis_index("s") * i32(NC) + lax.axis_index("c")
        pltpu.sync_copy(tok_hbm.at[wid], idx_v)

        def chunk_body(g, carry):
            cps = [
                pltpu.async_copy(
                    table_hbm.at[idx_v.at[g * i32(CHUNK_IDX) + i32(j)]],
                    rows_v.at[pl.ds(j * 128, 128)],
                    sem,
                )
                for j in range(CHUNK_IDX)
            ]
            for cp in cps:
                cp.wait()

            def node_body(n, c2):
                base = n * i32(T)
                for d in range(D // 16):
                    sl = pl.ds(d * 16, 16)
                    acc = rows_v[base, sl]
                    for t in range(1, T):
                        acc = acc + rows_v[base + i32(t), sl]
                    out_v[n, sl] = acc
                return c2

            lax.fori_loop(0, i32(CHUNK_NODES), node_body, i32(0))

            pltpu.sync_copy(
                out_v,
                out_hbm.at[pl.ds(wid * i32(NPW) + g * i32(CHUNK_NODES), CHUNK_NODES)],
            )
            return carry

        lax.fori_loop(0, i32(N_CHUNKS), chunk_body, i32(0))

    return k(buckets, tok)


def kernel(buckets, node_ids, token_ids):
    del node_ids
    tok = token_ids.astype(jnp.int32).reshape(NW, IDX_ROWS, 128)
    return _node_embed_sc(buckets, tok)


# speedup vs baseline: 9.3813x; 1.4297x over previous
import functools

import jax
import jax.numpy as jnp
from jax import lax
from jax.experimental import pallas as pl
from jax.experimental.pallas import tpu as pltpu
from jax.experimental.pallas import tpu_sc as plsc

D = 128
B = 16384
T = 20

NC = 2
NS = 16
NW = NC * NS
NPW = B // NW
ROWS_W = NPW * T

IDX_COLS = 80
IDX_ROWS = ROWS_W // IDX_COLS
CHUNK_NODES = 16
CHUNK_ROWS = CHUNK_NODES * T
IDX_PER_CHUNK = CHUNK_ROWS // IDX_COLS
N_CHUNKS = NPW // CHUNK_NODES
N_SUPER = N_CHUNKS // 2


def _node_embed_sc(buckets, tok):
    mesh = plsc.VectorSubcoreMesh(core_axis_name="c", subcore_axis_name="s")

    @functools.partial(
        pl.kernel,
        mesh=mesh,
        out_type=jax.ShapeDtypeStruct((B, D), jnp.float32),
        scratch_types=[
            pltpu.VMEM((IDX_ROWS, IDX_COLS), jnp.int32),
            pltpu.VMEM((CHUNK_ROWS, D), jnp.float32),
            pltpu.VMEM((CHUNK_ROWS, D), jnp.float32),
            pltpu.VMEM((CHUNK_NODES, D), jnp.float32),
            pltpu.SemaphoreType.DMA,
            pltpu.SemaphoreType.DMA,
        ],
    )
    def k(table_hbm, tok_hbm, out_hbm, idx_v, rows0, rows1, out_v, sem0, sem1):
        i32 = jnp.int32
        wid = lax.axis_index("s") * i32(NC) + lax.axis_index("c")
        pltpu.sync_copy(tok_hbm.at[wid], idx_v)

        def fire(c, buf, sem):
            for j in range(IDX_PER_CHUNK):
                pltpu.async_copy(
                    table_hbm.at[idx_v.at[c * i32(IDX_PER_CHUNK) + i32(j)]],
                    buf.at[pl.ds(j * IDX_COLS, IDX_COLS)],
                    sem,
                )

        def drain(c, buf, sem):
            for j in range(IDX_PER_CHUNK):
                pltpu.make_async_copy(
                    table_hbm.at[idx_v.at[c * i32(IDX_PER_CHUNK) + i32(j)]],
                    buf.at[pl.ds(j * IDX_COLS, IDX_COLS)],
                    sem,
                ).wait()

        def compute(c, buf):
            def node_body(n, c2):
                base = n * i32(T)
                for d in range(D // 16):
                    sl = pl.ds(d * 16, 16)
                    acc = buf[base, sl]
                    for t in range(1, T):
                        acc = acc + buf[base + i32(t), sl]
                    out_v[n, sl] = acc
                return c2

            lax.fori_loop(0, i32(CHUNK_NODES), node_body, i32(0))
            pltpu.sync_copy(
                out_v,
                out_hbm.at[pl.ds(wid * i32(NPW) + c * i32(CHUNK_NODES), CHUNK_NODES)],
            )

        fire(i32(0), rows0, sem0)

        def g_body(g, carry):
            c0 = g * i32(2)
            c1 = c0 + i32(1)
            fire(c1, rows1, sem1)
            drain(c0, rows0, sem0)
            compute(c0, rows0)

            @pl.when(g < i32(N_SUPER - 1))
            def _():
                fire(c0 + i32(2), rows0, sem0)

            drain(c1, rows1, sem1)
            compute(c1, rows1)
            return carry

        lax.fori_loop(0, i32(N_SUPER), g_body, i32(0))

    return k(buckets, tok)


def kernel(buckets, node_ids, token_ids):
    del node_ids
    tok = token_ids.astype(jnp.int32).reshape(NW, IDX_ROWS, IDX_COLS)
    return _node_embed_sc(buckets, tok)
